# baseline (speedup 1.0000x reference)
import jax
import jax.numpy as jnp
from jax import lax
from jax.experimental import pallas as pl
from jax.experimental.pallas import tpu as pltpu
from jax.experimental.pallas import tpu_sc as plsc

N = 10000
D = 128
E = 320000
NS = 16
L = 16
BLK = 112
NBUF = 3
GRAN = 12 * BLK
EPT = ((E + NS * GRAN - 1) // (NS * GRAN)) * GRAN + 4 * BLK
NBLK = EPT // BLK
EPAD = EPT * NS
ACC_ROWS = 10240
ZR = ACC_ROWS // NS
ZC = 80


def _sc_body(tab, src, dst, out_u, out_i, cnt_u, cnt_i, acc, *scr):
    cid = lax.axis_index("c")
    sid = lax.axis_index("s")
    r0 = sid * ZR
    base = cid * EPAD + sid * EPT
    scr = list(scr)
    idx_s = scr[0:4]
    idx_d = scr[4:8]
    rows = scr[8:11]
    gsem = scr[11:14]
    ssem = scr[14:17]
    isem = scr[17:21]
    dsem = scr[21:25]
    rows0 = rows[0]
    ones = rows[0]

    def start_idx(b, j):
        off = base + j * BLK
        pltpu.async_copy(src.at[pl.ds(off, BLK)], idx_s[b], isem[b])
        pltpu.async_copy(dst.at[pl.ds(off, BLK)], idx_d[b], dsem[b])

    def start_idx_d(b, j):
        off = base + j * BLK
        pltpu.async_copy(dst.at[pl.ds(off, BLK)], idx_d[b], dsem[b])

    def wait_idx_s(b):
        pltpu.make_async_copy(src.at[pl.ds(base, BLK)], idx_s[b], isem[b]).wait()

    def wait_idx_d(b):
        pltpu.make_async_copy(dst.at[pl.ds(base, BLK)], idx_d[b], dsem[b]).wait()

    def start_gather(r, i):
        pltpu.async_copy(tab.at[idx_s[i]], rows[r], gsem[r])

    def wait_gather(r):
        pltpu.make_async_copy(tab.at[idx_s[0]], rows[r], gsem[r]).wait()

    def start_scatter(r, i, src_buf):
        pltpu.async_copy(src_buf, acc.at[idx_d[i]], ssem[r], add=True)

    def wait_scatter(r, src_buf):
        pltpu.make_async_copy(src_buf, acc.at[idx_d[0]], ssem[r]).wait()

    pltpu.sync_copy(tab.at[pl.ds(2 * N, ZC)], rows0.at[pl.ds(0, ZC)])
    for k in range(ZR // ZC):
        pltpu.sync_copy(rows0.at[pl.ds(0, ZC)],
                        acc.at[pl.ds(r0 + k * ZC, ZC)])
    plsc.subcore_barrier()

    def p1_step(j, u3, u4, cold=False, nog=False, noi=False):
        r = u3
        r2 = (u3 + 2) % 3
        if not cold:
            wait_scatter(r2, rows[r2])
        if not nog:
            wait_idx_s((u4 + 2) % 4)
            start_gather(r2, (u4 + 2) % 4)
        wait_gather(r)
        wait_idx_d(u4)
        start_scatter(r, u4, rows[r])
        if not noi:
            start_idx((u4 + 3) % 4, j + 3)

    start_idx(0, 0)
    start_idx(1, 1)
    start_idx(2, 2)
    wait_idx_s(0)
    start_gather(0, 0)
    wait_idx_s(1)
    start_gather(1, 1)
    p1_step(0, 0, 0, cold=True)

    def outer1(g, carry):
        for u in range(12):
            j = 1 + 12 * g + u
            p1_step(j, (1 + u) % 3, (1 + u) % 4)
        return carry

    lax.fori_loop(0, (NBLK - 4) // 12, outer1, 0)
    p1_step(NBLK - 3, (NBLK - 3) % 3, (NBLK - 3) % 4, noi=True)
    p1_step(NBLK - 2, (NBLK - 2) % 3, (NBLK - 2) % 4, nog=True, noi=True)
    p1_step(NBLK - 1, (NBLK - 1) % 3, (NBLK - 1) % 4, nog=True, noi=True)
    wait_scatter((NBLK - 1) % 3, rows[(NBLK - 1) % 3])
    plsc.subcore_barrier()

    @pl.when(cid == 0)
    def _():
        pltpu.sync_copy(acc.at[pl.ds(r0, ZR)], out_u.at[pl.ds(r0, ZR)])

    @pl.when(cid == 1)
    def _():
        pltpu.sync_copy(acc.at[pl.ds(r0, ZR)], out_i.at[pl.ds(r0, ZR)])

    pltpu.sync_copy(tab.at[pl.ds(2 * N, ZC)], rows0.at[pl.ds(0, ZC)])
    for k in range(ZR // ZC):
        pltpu.sync_copy(rows0.at[pl.ds(0, ZC)],
                        acc.at[pl.ds(r0 + k * ZC, ZC)])
    pltpu.sync_copy(tab.at[pl.ds(2 * N + BLK, BLK)], ones)
    plsc.subcore_barrier()

    def p2_step(j, u, cold=False, tail=False):
        s = u % 2
        wait_idx_d(u)
        if not cold:
            wait_scatter(s, ones)
        start_scatter(s, u, ones)
        if not tail:
            start_idx_d((u + 2) % 4, j + 2)

    start_idx_d(0, 0)
    start_idx_d(1, 1)
    p2_step(0, 0, cold=True)
    p2_step(1, 1, cold=True)

    def outer2(g, carry):
        for u in range(4):
            j = 2 + 4 * g + u
            p2_step(j, (2 + u) % 4)
        return carry

    lax.fori_loop(0, (NBLK - 4) // 4, outer2, 0)
    p2_step(NBLK - 2, (NBLK - 2) % 4, tail=True)
    p2_step(NBLK - 1, (NBLK - 1) % 4, tail=True)
    wait_scatter(0, ones)
    wait_scatter(1, ones)
    plsc.subcore_barrier()

    @pl.when(cid == 0)
    def _():
        pltpu.sync_copy(acc.at[pl.ds(r0, ZR)], cnt_u.at[pl.ds(r0, ZR)])

    @pl.when(cid == 1)
    def _():
        pltpu.sync_copy(acc.at[pl.ds(r0, ZR)], cnt_i.at[pl.ds(r0, ZR)])


_sc_call = pl.kernel(
    _sc_body,
    out_type=(
        jax.ShapeDtypeStruct((ACC_ROWS, D), jnp.float32),
        jax.ShapeDtypeStruct((ACC_ROWS, D), jnp.float32),
        jax.ShapeDtypeStruct((ACC_ROWS, D), jnp.float32),
        jax.ShapeDtypeStruct((ACC_ROWS, D), jnp.float32),
    ),
    mesh=plsc.VectorSubcoreMesh(core_axis_name="c", subcore_axis_name="s"),
    scratch_types=(
        [pltpu.VMEM_SHARED((ACC_ROWS, D), jnp.float32)]
        + [pltpu.VMEM((BLK,), jnp.int32) for _ in range(8)]
        + [pltpu.VMEM((BLK, D), jnp.float32) for _ in range(3)]
        + [pltpu.SemaphoreType.DMA for _ in range(14)]
    ),
)


def _tc_body(x_ref, agg_ref, cnt_ref, ws_ref, wc_ref, b_ref, y_ref, p_ref):
    i = pl.program_id(0)
    x = x_ref[...]
    h = jnp.dot(x, ws_ref[...], preferred_element_type=jnp.float32)
    h += jnp.dot(agg_ref[...], wc_ref[...], preferred_element_type=jnp.float32)
    h += b_ref[...]
    mask = (cnt_ref[:, :1] > 0).astype(jnp.float32)
    delta = jnp.maximum(h, 0.0) * mask
    y_ref[...] = x + delta

    @pl.when(i == 0)
    def _():
        p_ref[...] = jnp.zeros_like(p_ref)

    lanes = lax.broadcasted_iota(jnp.int32, (1, D), 1)
    contrib = (jnp.where(lanes == 0, jnp.sum(delta * delta), 0.0)
               + jnp.where(lanes == 1, jnp.sum(mask), 0.0))
    p_ref[...] += contrib


TCB = 400


def _tc_call(x, agg, cnt, ws_t, wc_t, b):
    return pl.pallas_call(
        _tc_body,
        grid=(N // TCB,),
        in_specs=[
            pl.BlockSpec((TCB, D), lambda i: (i, 0)),
            pl.BlockSpec((TCB, D), lambda i: (i, 0)),
            pl.BlockSpec((TCB, D), lambda i: (i, 0)),
            pl.BlockSpec((D, D), lambda i: (0, 0)),
            pl.BlockSpec((D, D), lambda i: (0, 0)),
            pl.BlockSpec((1, D), lambda i: (0, 0)),
        ],
        out_specs=(
            pl.BlockSpec((TCB, D), lambda i: (i, 0)),
            pl.BlockSpec((1, D), lambda i: (0, 0)),
        ),
        out_shape=(
            jax.ShapeDtypeStruct((N, D), jnp.float32),
            jax.ShapeDtypeStruct((1, D), jnp.float32),
        ),
    )(x, agg, cnt, ws_t, wc_t, b)


def kernel(xu_t_minus, xi_t_minus, adj_ins_i2u, adj_ins_u2i,
           W_uu, b_uu, W_ii, b_ii, W_ui, W_iu):
    dst_u = adj_ins_i2u[0].astype(jnp.int32)
    src_i = adj_ins_i2u[1].astype(jnp.int32)
    dst_i = adj_ins_u2i[0].astype(jnp.int32)
    src_u = adj_ins_u2i[1].astype(jnp.int32)

    tab = jnp.concatenate(
        [xi_t_minus, xu_t_minus, jnp.zeros((BLK, D), jnp.float32),
         jnp.ones((BLK, D), jnp.float32)], axis=0)

    pad = EPAD - E
    src = jnp.concatenate([
        jnp.pad(src_i, (0, pad)),
        jnp.pad(src_u, (0, pad)) + N,
    ])
    dst = jnp.concatenate([
        jnp.pad(dst_u, (0, pad), constant_values=N),
        jnp.pad(dst_i, (0, pad), constant_values=N),
    ])

    agg_u, agg_i, cnt_u, cnt_i = _sc_call(tab, src, dst)
    y_u, p_u = _tc_call(xu_t_minus, agg_u, cnt_u, W_uu.T, W_iu.T, b_uu[None, :])
    y_i, p_i = _tc_call(xi_t_minus, agg_i, cnt_i, W_ii.T, W_ui.T, b_ii[None, :])

    loss = p_u[0, 0] / p_u[0, 1] + p_i[0, 0] / p_i[0, 1]
    return (y_u, y_i, loss)


# speedup vs baseline: 1.4811x; 1.4811x over previous
import jax
import jax.numpy as jnp
from jax import lax
from jax.experimental import pallas as pl
from jax.experimental.pallas import tpu as pltpu
from jax.experimental.pallas import tpu_sc as plsc

N = 10000
D = 128
E = 320000
NS = 16
L = 16
BLK = 112
NBUF = 3
GRAN = NBUF * BLK
EPT = ((E + NS * GRAN - 1) // (NS * GRAN)) * GRAN
NBLK = EPT // BLK
EPAD = EPT * NS
ACC_ROWS = 10240
ZR = ACC_ROWS // NS
ZC = 80


def _sc_body(tab, src, dst, out_u, out_i, cnt_u, cnt_i, acc, *scr):
    cid = lax.axis_index("c")
    sid = lax.axis_index("s")
    r0 = sid * ZR
    base = cid * EPAD + sid * EPT
    scr = list(scr)
    idx_s = scr[0:NBUF]
    idx_d = scr[NBUF:2 * NBUF]
    rows = scr[2 * NBUF:3 * NBUF]
    gsem = scr[3 * NBUF:4 * NBUF]
    ssem = scr[4 * NBUF:5 * NBUF]
    isem = scr[5 * NBUF:6 * NBUF]
    dsem = scr[6 * NBUF:7 * NBUF]
    rows0 = rows[0]
    ones = rows[0]

    def start_idx(b, j):
        off = base + j * BLK
        pltpu.async_copy(src.at[pl.ds(off, BLK)], idx_s[b], isem[b])
        pltpu.async_copy(dst.at[pl.ds(off, BLK)], idx_d[b], dsem[b])

    def start_idx_d(b, j):
        off = base + j * BLK
        pltpu.async_copy(dst.at[pl.ds(off, BLK)], idx_d[b], dsem[b])

    def wait_idx_s(b):
        pltpu.make_async_copy(src.at[pl.ds(base, BLK)], idx_s[b], isem[b]).wait()

    def wait_idx_d(b):
        pltpu.make_async_copy(dst.at[pl.ds(base, BLK)], idx_d[b], dsem[b]).wait()

    def start_gather(r, i):
        pltpu.async_copy(tab.at[idx_s[i]], rows[r], gsem[r])

    def wait_gather(r):
        pltpu.make_async_copy(tab.at[idx_s[0]], rows[r], gsem[r]).wait()

    def start_scatter(r, i, src_buf):
        pltpu.async_copy(src_buf, acc.at[idx_d[i]], ssem[r], add=True)

    def wait_scatter(r, src_buf):
        pltpu.make_async_copy(src_buf, acc.at[idx_d[0]], ssem[r]).wait()

    pltpu.sync_copy(tab.at[pl.ds(2 * N, ZC)], rows0.at[pl.ds(0, ZC)])
    for k in range(ZR // ZC):
        pltpu.sync_copy(rows0.at[pl.ds(0, ZC)],
                        acc.at[pl.ds(r0 + k * ZC, ZC)])
    plsc.subcore_barrier()

    for b in range(NBUF):
        start_idx(b, b)
    for b in range(NBUF):
        wait_idx_s(b)
        start_gather(b, b)

    def outer1(g, carry):
        j0 = g * NBUF
        for b in range(NBUF):
            wait_gather(b)
            wait_idx_d(b)
            start_scatter(b, b, rows[b])
        for b in range(NBUF):
            wait_scatter(b, rows[b])
            start_idx(b, j0 + NBUF + b)
        for b in range(NBUF):
            wait_idx_s(b)
            start_gather(b, b)
        return carry

    lax.fori_loop(0, NBLK // NBUF - 1, outer1, 0)
    for b in range(NBUF):
        wait_gather(b)
        wait_idx_d(b)
        start_scatter(b, b, rows[b])
    for b in range(NBUF):
        wait_scatter(b, rows[b])
    plsc.subcore_barrier()

    @pl.when(cid == 0)
    def _():
        pltpu.sync_copy(acc.at[pl.ds(r0, ZR)], out_u.at[pl.ds(r0, ZR)])

    @pl.when(cid == 1)
    def _():
        pltpu.sync_copy(acc.at[pl.ds(r0, ZR)], out_i.at[pl.ds(r0, ZR)])

    pltpu.sync_copy(tab.at[pl.ds(2 * N, ZC)], rows0.at[pl.ds(0, ZC)])
    for k in range(ZR // ZC):
        pltpu.sync_copy(rows0.at[pl.ds(0, ZC)],
                        acc.at[pl.ds(r0 + k * ZC, ZC)])
    pltpu.sync_copy(tab.at[pl.ds(2 * N + BLK, BLK)], ones)
    plsc.subcore_barrier()

    for b in range(NBUF):
        start_idx_d(b, b)

    def outer2(g, carry):
        j0 = g * NBUF
        for b in range(NBUF):
            wait_idx_d(b)
            start_scatter(b, b, ones)
        for b in range(NBUF):
            wait_scatter(b, ones)
            start_idx_d(b, j0 + NBUF + b)
        return carry

    lax.fori_loop(0, NBLK // NBUF - 1, outer2, 0)
    for b in range(NBUF):
        wait_idx_d(b)
        start_scatter(b, b, ones)
    for b in range(NBUF):
        wait_scatter(b, ones)
    plsc.subcore_barrier()

    @pl.when(cid == 0)
    def _():
        pltpu.sync_copy(acc.at[pl.ds(r0, ZR)], cnt_u.at[pl.ds(r0, ZR)])

    @pl.when(cid == 1)
    def _():
        pltpu.sync_copy(acc.at[pl.ds(r0, ZR)], cnt_i.at[pl.ds(r0, ZR)])


_sc_call = pl.kernel(
    _sc_body,
    out_type=(
        jax.ShapeDtypeStruct((ACC_ROWS, D), jnp.float32),
        jax.ShapeDtypeStruct((ACC_ROWS, D), jnp.float32),
        jax.ShapeDtypeStruct((ACC_ROWS, D), jnp.float32),
        jax.ShapeDtypeStruct((ACC_ROWS, D), jnp.float32),
    ),
    mesh=plsc.VectorSubcoreMesh(core_axis_name="c", subcore_axis_name="s"),
    scratch_types=(
        [pltpu.VMEM_SHARED((ACC_ROWS, D), jnp.float32)]
        + [pltpu.VMEM((BLK,), jnp.int32) for _ in range(2 * NBUF)]
        + [pltpu.VMEM((BLK, D), jnp.float32) for _ in range(NBUF)]
        + [pltpu.SemaphoreType.DMA for _ in range(4 * NBUF)]
    ),
)


def _tc_body(x_ref, agg_ref, cnt_ref, ws_ref, wc_ref, b_ref, y_ref, p_ref):
    i = pl.program_id(0)
    x = x_ref[...]
    h = jnp.dot(x, ws_ref[...], preferred_element_type=jnp.float32)
    h += jnp.dot(agg_ref[...], wc_ref[...], preferred_element_type=jnp.float32)
    h += b_ref[...]
    mask = (cnt_ref[:, :1] > 0).astype(jnp.float32)
    delta = jnp.maximum(h, 0.0) * mask
    y_ref[...] = x + delta

    @pl.when(i == 0)
    def _():
        p_ref[...] = jnp.zeros_like(p_ref)

    lanes = lax.broadcasted_iota(jnp.int32, (1, D), 1)
    contrib = (jnp.where(lanes == 0, jnp.sum(delta * delta), 0.0)
               + jnp.where(lanes == 1, jnp.sum(mask), 0.0))
    p_ref[...] += contrib


TCB = 400


def _tc_call(x, agg, cnt, ws_t, wc_t, b):
    return pl.pallas_call(
        _tc_body,
        grid=(N // TCB,),
        in_specs=[
            pl.BlockSpec((TCB, D), lambda i: (i, 0)),
            pl.BlockSpec((TCB, D), lambda i: (i, 0)),
            pl.BlockSpec((TCB, D), lambda i: (i, 0)),
            pl.BlockSpec((D, D), lambda i: (0, 0)),
            pl.BlockSpec((D, D), lambda i: (0, 0)),
            pl.BlockSpec((1, D), lambda i: (0, 0)),
        ],
        out_specs=(
            pl.BlockSpec((TCB, D), lambda i: (i, 0)),
            pl.BlockSpec((1, D), lambda i: (0, 0)),
        ),
        out_shape=(
            jax.ShapeDtypeStruct((N, D), jnp.float32),
            jax.ShapeDtypeStruct((1, D), jnp.float32),
        ),
    )(x, agg, cnt, ws_t, wc_t, b)


def kernel(xu_t_minus, xi_t_minus, adj_ins_i2u, adj_ins_u2i,
           W_uu, b_uu, W_ii, b_ii, W_ui, W_iu):
    dst_u = adj_ins_i2u[0].astype(jnp.int32)
    src_i = adj_ins_i2u[1].astype(jnp.int32)
    dst_i = adj_ins_u2i[0].astype(jnp.int32)
    src_u = adj_ins_u2i[1].astype(jnp.int32)

    tab = jnp.concatenate(
        [xi_t_minus, xu_t_minus, jnp.zeros((BLK, D), jnp.float32),
         jnp.ones((BLK, D), jnp.float32)], axis=0)

    pad = EPAD - E
    src = jnp.concatenate([
        jnp.pad(src_i, (0, pad)),
        jnp.pad(src_u, (0, pad)) + N,
    ])
    dst = jnp.concatenate([
        jnp.pad(dst_u, (0, pad), constant_values=N),
        jnp.pad(dst_i, (0, pad), constant_values=N),
    ])

    agg_u, agg_i, cnt_u, cnt_i = _sc_call(tab, src, dst)
    y_u, p_u = _tc_call(xu_t_minus, agg_u, cnt_u, W_uu.T, W_iu.T, b_uu[None, :])
    y_i, p_i = _tc_call(xi_t_minus, agg_i, cnt_i, W_ii.T, W_ui.T, b_ii[None, :])

    loss = p_u[0, 0] / p_u[0, 1] + p_i[0, 0] / p_i[0, 1]
    return (y_u, y_i, loss)


# speedup vs baseline: 1.7149x; 1.1579x over previous
import jax
import jax.numpy as jnp
from jax import lax
from jax.experimental import pallas as pl
from jax.experimental.pallas import tpu as pltpu
from jax.experimental.pallas import tpu_sc as plsc

N = 10000
D = 128
E = 320000
NS = 16
L = 16
BLK = 88
NBUF = 4
GRAN = NBUF * BLK
EPT = ((E + NS * GRAN - 1) // (NS * GRAN)) * GRAN
NBLK = EPT // BLK
EPAD = EPT * NS
ACC_ROWS = 10240
ZR = ACC_ROWS // NS
ZC = 80


def _sc_body(tab, src, dst, out_u, out_i, cnt_u, cnt_i, acc, *scr):
    cid = lax.axis_index("c")
    sid = lax.axis_index("s")
    r0 = sid * ZR
    base = cid * EPAD + sid * EPT
    scr = list(scr)
    idx_s = scr[0:NBUF]
    idx_d = scr[NBUF:2 * NBUF]
    rows = scr[2 * NBUF:3 * NBUF]
    gsem = scr[3 * NBUF:4 * NBUF]
    ssem = scr[4 * NBUF:5 * NBUF]
    isem = scr[5 * NBUF:6 * NBUF]
    dsem = scr[6 * NBUF:7 * NBUF]
    rows0 = rows[0]
    ones = rows[0]

    def start_idx(b, j):
        off = base + j * BLK
        pltpu.async_copy(src.at[pl.ds(off, BLK)], idx_s[b], isem[b])
        pltpu.async_copy(dst.at[pl.ds(off, BLK)], idx_d[b], dsem[b])

    def start_idx_d(b, j):
        off = base + j * BLK
        pltpu.async_copy(dst.at[pl.ds(off, BLK)], idx_d[b], dsem[b])

    def wait_idx_s(b):
        pltpu.make_async_copy(src.at[pl.ds(base, BLK)], idx_s[b], isem[b]).wait()

    def wait_idx_d(b):
        pltpu.make_async_copy(dst.at[pl.ds(base, BLK)], idx_d[b], dsem[b]).wait()

    def start_gather(r, i):
        pltpu.async_copy(tab.at[idx_s[i]], rows[r], gsem[r])

    def wait_gather(r):
        pltpu.make_async_copy(tab.at[idx_s[0]], rows[r], gsem[r]).wait()

    def start_scatter(r, i, src_buf):
        pltpu.async_copy(src_buf, acc.at[idx_d[i]], ssem[r], add=True)

    def wait_scatter(r, src_buf):
        pltpu.make_async_copy(src_buf, acc.at[idx_d[0]], ssem[r]).wait()

    pltpu.sync_copy(tab.at[pl.ds(2 * N, ZC)], rows0.at[pl.ds(0, ZC)])
    for k in range(ZR // ZC):
        pltpu.sync_copy(rows0.at[pl.ds(0, ZC)],
                        acc.at[pl.ds(r0 + k * ZC, ZC)])
    plsc.subcore_barrier()

    for b in range(NBUF):
        start_idx(b, b)
    for b in range(NBUF):
        wait_idx_s(b)
        start_gather(b, b)

    def outer1(g, carry):
        j0 = g * NBUF
        for b in range(NBUF):
            wait_gather(b)
            wait_idx_d(b)
            start_scatter(b, b, rows[b])
        for b in range(NBUF):
            wait_scatter(b, rows[b])
            start_idx(b, j0 + NBUF + b)
        for b in range(NBUF):
            wait_idx_s(b)
            start_gather(b, b)
        return carry

    lax.fori_loop(0, NBLK // NBUF - 1, outer1, 0)
    for b in range(NBUF):
        wait_gather(b)
        wait_idx_d(b)
        start_scatter(b, b, rows[b])
    for b in range(NBUF):
        wait_scatter(b, rows[b])
    plsc.subcore_barrier()

    @pl.when(cid == 0)
    def _():
        pltpu.sync_copy(acc.at[pl.ds(r0, ZR)], out_u.at[pl.ds(r0, ZR)])

    @pl.when(cid == 1)
    def _():
        pltpu.sync_copy(acc.at[pl.ds(r0, ZR)], out_i.at[pl.ds(r0, ZR)])

    pltpu.sync_copy(tab.at[pl.ds(2 * N, ZC)], rows0.at[pl.ds(0, ZC)])
    for k in range(ZR // ZC):
        pltpu.sync_copy(rows0.at[pl.ds(0, ZC)],
                        acc.at[pl.ds(r0 + k * ZC, ZC)])
    pltpu.sync_copy(tab.at[pl.ds(2 * N + BLK, BLK)], ones)
    plsc.subcore_barrier()

    for b in range(NBUF):
        start_idx_d(b, b)

    def outer2(g, carry):
        j0 = g * NBUF
        for b in range(NBUF):
            wait_idx_d(b)
            start_scatter(b, b, ones)
        for b in range(NBUF):
            wait_scatter(b, ones)
            start_idx_d(b, j0 + NBUF + b)
        return carry

    lax.fori_loop(0, NBLK // NBUF - 1, outer2, 0)
    for b in range(NBUF):
        wait_idx_d(b)
        start_scatter(b, b, ones)
    for b in range(NBUF):
        wait_scatter(b, ones)
    plsc.subcore_barrier()

    @pl.when(cid == 0)
    def _():
        pltpu.sync_copy(acc.at[pl.ds(r0, ZR)], cnt_u.at[pl.ds(r0, ZR)])

    @pl.when(cid == 1)
    def _():
        pltpu.sync_copy(acc.at[pl.ds(r0, ZR)], cnt_i.at[pl.ds(r0, ZR)])


_sc_call = pl.kernel(
    _sc_body,
    out_type=(
        jax.ShapeDtypeStruct((ACC_ROWS, D), jnp.float32),
        jax.ShapeDtypeStruct((ACC_ROWS, D), jnp.float32),
        jax.ShapeDtypeStruct((ACC_ROWS, D), jnp.float32),
        jax.ShapeDtypeStruct((ACC_ROWS, D), jnp.float32),
    ),
    mesh=plsc.VectorSubcoreMesh(core_axis_name="c", subcore_axis_name="s"),
    scratch_types=(
        [pltpu.VMEM_SHARED((ACC_ROWS, D), jnp.float32)]
        + [pltpu.VMEM((BLK,), jnp.int32) for _ in range(2 * NBUF)]
        + [pltpu.VMEM((BLK, D), jnp.float32) for _ in range(NBUF)]
        + [pltpu.SemaphoreType.DMA for _ in range(4 * NBUF)]
    ),
)


def _tc_body(x_ref, agg_ref, cnt_ref, ws_ref, wc_ref, b_ref, y_ref, p_ref):
    i = pl.program_id(0)
    x = x_ref[...]
    h = jnp.dot(x, ws_ref[...], preferred_element_type=jnp.float32)
    h += jnp.dot(agg_ref[...], wc_ref[...], preferred_element_type=jnp.float32)
    h += b_ref[...]
    mask = (cnt_ref[:, :1] > 0).astype(jnp.float32)
    delta = jnp.maximum(h, 0.0) * mask
    y_ref[...] = x + delta

    @pl.when(i == 0)
    def _():
        p_ref[...] = jnp.zeros_like(p_ref)

    lanes = lax.broadcasted_iota(jnp.int32, (1, D), 1)
    contrib = (jnp.where(lanes == 0, jnp.sum(delta * delta), 0.0)
               + jnp.where(lanes == 1, jnp.sum(mask), 0.0))
    p_ref[...] += contrib


TCB = 400


def _tc_call(x, agg, cnt, ws_t, wc_t, b):
    return pl.pallas_call(
        _tc_body,
        grid=(N // TCB,),
        in_specs=[
            pl.BlockSpec((TCB, D), lambda i: (i, 0)),
            pl.BlockSpec((TCB, D), lambda i: (i, 0)),
            pl.BlockSpec((TCB, D), lambda i: (i, 0)),
            pl.BlockSpec((D, D), lambda i: (0, 0)),
            pl.BlockSpec((D, D), lambda i: (0, 0)),
            pl.BlockSpec((1, D), lambda i: (0, 0)),
        ],
        out_specs=(
            pl.BlockSpec((TCB, D), lambda i: (i, 0)),
            pl.BlockSpec((1, D), lambda i: (0, 0)),
        ),
        out_shape=(
            jax.ShapeDtypeStruct((N, D), jnp.float32),
            jax.ShapeDtypeStruct((1, D), jnp.float32),
        ),
    )(x, agg, cnt, ws_t, wc_t, b)


def kernel(xu_t_minus, xi_t_minus, adj_ins_i2u, adj_ins_u2i,
           W_uu, b_uu, W_ii, b_ii, W_ui, W_iu):
    dst_u = adj_ins_i2u[0].astype(jnp.int32)
    src_i = adj_ins_i2u[1].astype(jnp.int32)
    dst_i = adj_ins_u2i[0].astype(jnp.int32)
    src_u = adj_ins_u2i[1].astype(jnp.int32)

    tab = jnp.concatenate(
        [xi_t_minus, xu_t_minus, jnp.zeros((BLK, D), jnp.float32),
         jnp.ones((BLK, D), jnp.float32)], axis=0)

    pad = EPAD - E
    src = jnp.concatenate([
        jnp.pad(src_i, (0, pad)),
        jnp.pad(src_u, (0, pad)) + N,
    ])
    dst = jnp.concatenate([
        jnp.pad(dst_u, (0, pad), constant_values=N),
        jnp.pad(dst_i, (0, pad), constant_values=N),
    ])

    agg_u, agg_i, cnt_u, cnt_i = _sc_call(tab, src, dst)
    y_u, p_u = _tc_call(xu_t_minus, agg_u, cnt_u, W_uu.T, W_iu.T, b_uu[None, :])
    y_i, p_i = _tc_call(xi_t_minus, agg_i, cnt_i, W_ii.T, W_ui.T, b_ii[None, :])

    loss = p_u[0, 0] / p_u[0, 1] + p_i[0, 0] / p_i[0, 1]
    return (y_u, y_i, loss)
